# baseline (speedup 1.0000x reference)
import functools

import jax
import jax.numpy as jnp
from jax import lax
from jax.experimental import pallas as pl
from jax.experimental.pallas import tpu as pltpu
from jax.experimental.pallas import tpu_sc as plsc

N = 10000
E = 320000
G = 64
D_IN = 128
D_H = 64
HEADS = 2
D_OUT = 64

RB = 400
NBLK = N // RB
NCORES = 2
NSUB = 16
K = 128
EV = E + N
EP = 335872
NCHT = EP // K
RPT = N // NSUB

D = 64
W = 80


def _proj_body(x_ref, wl_ref, bl_ref, wr_ref, br_ref, xl_ref, xr_ref):
    xb = x_ref[...]
    xl_ref[...] = jnp.dot(xb, wl_ref[...], preferred_element_type=jnp.float32) + bl_ref[...]
    xr_ref[...] = jnp.dot(xb, wr_ref[...], preferred_element_type=jnp.float32) + br_ref[...]


def _proj_call(xp, Wl, bl, Wr, br, d_in, d_out):
    return pl.pallas_call(
        _proj_body,
        grid=(NBLK,),
        in_specs=[
            pl.BlockSpec((RB, d_in), lambda i: (i, 0)),
            pl.BlockSpec((d_in, d_out), lambda i: (0, 0)),
            pl.BlockSpec((1, d_out), lambda i: (0, 0)),
            pl.BlockSpec((d_in, d_out), lambda i: (0, 0)),
            pl.BlockSpec((1, d_out), lambda i: (0, 0)),
        ],
        out_specs=[
            pl.BlockSpec((RB, d_out), lambda i: (i, 0)),
            pl.BlockSpec((RB, d_out), lambda i: (i, 0)),
        ],
        out_shape=[
            jax.ShapeDtypeStruct((N, d_out), jnp.float32),
            jax.ShapeDtypeStruct((N, d_out), jnp.float32),
        ],
    )(xp, Wl, bl, Wr, br)


def _mid_body(a0_ref, a1_ref, b1_ref, wl_ref, bl_ref, wr_ref, br_ref,
              xl_ref, xr_ref):
    s0 = a0_ref[...]
    s1 = a1_ref[...]
    h0 = s0[:, 0:64] / (s0[:, 64:65] + 1e-16)
    h1 = s1[:, 0:64] / (s1[:, 64:65] + 1e-16)
    h = jnp.concatenate([h0, h1], axis=1) + b1_ref[...]
    h = jnp.where(h > 0, h, jnp.exp(h) - 1.0)
    xl_ref[...] = jnp.dot(h, wl_ref[...], preferred_element_type=jnp.float32) + bl_ref[...]
    xr_ref[...] = jnp.dot(h, wr_ref[...], preferred_element_type=jnp.float32) + br_ref[...]


def _mid_call(accA, accB, b1, Wl2, bl2, Wr2, br2):
    return pl.pallas_call(
        _mid_body,
        grid=(NBLK,),
        in_specs=[
            pl.BlockSpec((RB, W), lambda i: (i, 0)),
            pl.BlockSpec((RB, W), lambda i: (i, 0)),
            pl.BlockSpec((1, 128), lambda i: (0, 0)),
            pl.BlockSpec((128, 64), lambda i: (0, 0)),
            pl.BlockSpec((1, 64), lambda i: (0, 0)),
            pl.BlockSpec((128, 64), lambda i: (0, 0)),
            pl.BlockSpec((1, 64), lambda i: (0, 0)),
        ],
        out_specs=[
            pl.BlockSpec((RB, 64), lambda i: (i, 0)),
            pl.BlockSpec((RB, 64), lambda i: (i, 0)),
        ],
        out_shape=[
            jax.ShapeDtypeStruct((N, 64), jnp.float32),
            jax.ShapeDtypeStruct((N, 64), jnp.float32),
        ],
    )(accA, accB, b1, Wl2, bl2, Wr2, br2)


def _pool_body(a_ref, b_ref, b2_ref, bat_ref, wlin_ref, blin_ref,
               out_ref, sums_ref, cnt_ref):
    i = pl.program_id(0)
    s = a_ref[...] + b_ref[...]
    h = s[:, 0:64] / (s[:, 64:65] + 1e-16) + b2_ref[...]
    h = jnp.where(h > 0, h, jnp.exp(h) - 1.0)
    gid = lax.broadcasted_iota(jnp.int32, (1, G), 1)
    oh = (bat_ref[...] == gid).astype(jnp.float32)

    @pl.when(i == 0)
    def _():
        sums_ref[...] = jnp.zeros_like(sums_ref)
        cnt_ref[...] = jnp.zeros_like(cnt_ref)

    dn = (((0,), (0,)), ((), ()))
    sums_ref[...] += lax.dot_general(oh, h, dn, preferred_element_type=jnp.float32)
    cnt_ref[...] += lax.dot_general(oh, jnp.ones((RB, 1), jnp.float32), dn,
                                    preferred_element_type=jnp.float32)

    @pl.when(i == NBLK - 1)
    def _():
        pooled = sums_ref[...] / jnp.maximum(cnt_ref[...], 1.0)
        out_ref[...] = jnp.dot(pooled, wlin_ref[...],
                               preferred_element_type=jnp.float32) + blin_ref[...]


def _pool_call(accA, accB, b2, batp, Wlin, blin):
    return pl.pallas_call(
        _pool_body,
        grid=(NBLK,),
        in_specs=[
            pl.BlockSpec((RB, W), lambda i: (i, 0)),
            pl.BlockSpec((RB, W), lambda i: (i, 0)),
            pl.BlockSpec((1, 64), lambda i: (0, 0)),
            pl.BlockSpec((RB, 1), lambda i: (i, 0)),
            pl.BlockSpec((64, 2), lambda i: (0, 0)),
            pl.BlockSpec((1, 2), lambda i: (0, 0)),
        ],
        out_specs=pl.BlockSpec((G, 2), lambda i: (0, 0)),
        out_shape=jax.ShapeDtypeStruct((G, 2), jnp.float32),
        scratch_shapes=[
            pltpu.VMEM((G, 64), jnp.float32),
            pltpu.VMEM((G, 1), jnp.float32),
        ],
    )(accA, accB, b2, batp, Wlin, blin)


def _make_edge_kernel(split):
    NJ = D // 16
    ept = EP // NSUB if split else EP // (NCORES * NSUB)
    nch = ept // K
    mesh = plsc.VectorSubcoreMesh(core_axis_name="c", subcore_axis_name="s")

    @functools.partial(
        pl.kernel,
        mesh=mesh,
        compiler_params=pltpu.CompilerParams(
            needs_layout_passes=False, use_tc_tiling_on_sc=False),
        out_type=jax.ShapeDtypeStruct((NCORES, N, W), jnp.float32),
        scratch_types=[
            pltpu.VMEM((2, K), jnp.int32),
            pltpu.VMEM((2, K), jnp.int32),
            pltpu.VMEM((K,), jnp.int32),
            pltpu.VMEM((K,), jnp.int32),
            pltpu.VMEM((K,), jnp.int32),
            pltpu.VMEM((K,), jnp.int32),
            pltpu.VMEM((K, D), jnp.float32),
            pltpu.VMEM((K, D), jnp.float32),
            pltpu.VMEM((K, D), jnp.float32),
            pltpu.VMEM((K, D), jnp.float32),
            pltpu.VMEM((K,), jnp.int32),
            pltpu.VMEM((K,), jnp.int32),
            pltpu.VMEM((K, W), jnp.float32),
            pltpu.VMEM((K, W), jnp.float32),
            pltpu.VMEM((D * (2 if split else 1),), jnp.float32),
            pltpu.VMEM_SHARED((N, W), jnp.float32),
            pltpu.SemaphoreType.DMA,
            pltpu.SemaphoreType.DMA,
            pltpu.SemaphoreType.DMA,
            pltpu.SemaphoreType.DMA,
            pltpu.SemaphoreType.DMA,
            pltpu.SemaphoreType.DMA,
        ],
    )
    def edge_kernel(xl_hbm, xr_hbm, sd_hbm, att_hbm, out_hbm,
                    sdA, sdB, sgA, dgA, sgB, dgB, xlA, xrA, xlB, xrB,
                    sciA, sciB, conA, conB, att_v, acc_sh,
                    semA1, semA2, semB1, semB2, semSA, semSB):
        cid = lax.axis_index("c")
        sid = lax.axis_index("s")

        zeros16 = jnp.zeros((16,), jnp.float32)

        zi16 = jnp.zeros((16,), jnp.int32)

        def _zrow(r, carry):
            for off in list(range(0, W - 16, 16)) + [W - 16]:
                conA[r, pl.ds(off, 16)] = zeros16
                conB[r, pl.ds(off, 16)] = zeros16
            return carry
        lax.fori_loop(0, K, _zrow, 0)
        for j in range(K // 16):
            sciA[pl.ds(16 * j, 16)] = zi16
            sciB[pl.ds(16 * j, 16)] = zi16
        for t in range(4):
            pltpu.sync_copy(conA, acc_sh.at[pl.ds(sid * RPT + t * K, K)])
        pltpu.sync_copy(conA.at[pl.ds(0, RPT - 4 * K)],
                        acc_sh.at[pl.ds(sid * RPT + 4 * K, RPT - 4 * K)])
        plsc.subcore_barrier()
        pltpu.async_copy(conA, acc_sh.at[sciA], semSA, add=True)
        pltpu.async_copy(conB, acc_sh.at[sciB], semSB, add=True)

        pltpu.sync_copy(att_hbm, att_v)
        aoff = cid * D if split else 0
        attv = [att_v[pl.ds(aoff + 16 * j, 16)] for j in range(NJ)]
        lane = lax.iota(jnp.int32, 16)

        wchunk = (sid if split else cid * NSUB + sid) * nch

        bufs = (
            (sdA, sgA, dgA, xlA, xrA, sciA, conA, semA1, semA2, semSA),
            (sdB, sgB, dgB, xlB, xrB, sciB, conB, semB1, semB2, semSB),
        )

        def _fetch(gch, p):
            sd, sg, dg, xlr, xrr, sci, con, s1, s2, ss = bufs[p]
            pltpu.sync_copy(sd_hbm.at[gch], sd)
            if split:
                for j in range(K // 16):
                    sv = sd[0, pl.ds(16 * j, 16)]
                    dv = sd[1, pl.ds(16 * j, 16)]
                    sg[pl.ds(16 * j, 16)] = sv + sv + cid
                    dg[pl.ds(16 * j, 16)] = dv + dv + cid
                pltpu.async_copy(xl_hbm.at[sg], xlr, s1)
                pltpu.async_copy(xr_hbm.at[dg], xrr, s2)
            else:
                pltpu.async_copy(xl_hbm.at[sd.at[0]], xlr, s1)
                pltpu.async_copy(xr_hbm.at[sd.at[1]], xrr, s2)

        def _wait(p):
            sd, sg, dg, xlr, xrr, sci, con, s1, s2, ss = bufs[p]
            src_ref = xl_hbm.at[sg] if split else xl_hbm.at[sd.at[0]]
            pltpu.make_async_copy(src_ref, xlr, s1).wait()
            src_ref = xr_hbm.at[dg] if split else xr_hbm.at[sd.at[1]]
            pltpu.make_async_copy(src_ref, xrr, s2).wait()

        def _make_edge(xlr, xrr, con):
            def _edge(e, base):
                valid = jnp.full((16,), base + e < EV)
                xlv = [xlr[e, pl.ds(16 * j, 16)] for j in range(NJ)]
                xrv = [xrr[e, pl.ds(16 * j, 16)] for j in range(NJ)]
                pr = []
                for j in range(NJ):
                    z = xlv[j] + xrv[j]
                    pr.append(jnp.maximum(z, 0.2 * z) * attv[j])
                sacc = pr[0]
                for j in range(1, NJ):
                    sacc = sacc + pr[j]
                a = plsc.cumsum(sacc)[15]
                wv = jnp.exp(jnp.full((16,), a, jnp.float32))
                wv = jnp.where(valid, wv, zeros16)
                for j in range(NJ):
                    con[e, pl.ds(16 * j, 16)] = xlv[j] * wv
                con[e, pl.ds(D, 16)] = jnp.where(lane == 0, wv, zeros16)
                return base
            return _edge

        ebase = wchunk * K
        _fetch(wchunk, 0)

        def _pair(g, carry):
            for p in (0, 1):
                ch = 2 * g + p
                _fetch(wchunk + ch + 1, 1 - p)
                _wait(p)
                sd, sg, dg, xlr, xrr, sci, con, s1, s2, ss = bufs[p]
                pltpu.make_async_copy(con, acc_sh.at[sci], ss).wait()
                for j in range(K // 16):
                    sci[pl.ds(16 * j, 16)] = sd[1, pl.ds(16 * j, 16)]
                lax.fori_loop(0, K, _make_edge(xlr, xrr, con),
                              ebase + ch * K, unroll=8)
                pltpu.async_copy(con, acc_sh.at[sci], ss, add=True)
            return carry

        lax.fori_loop(0, nch // 2, _pair, 0)
        _wait(0)
        pltpu.make_async_copy(conA, acc_sh.at[sciA], semSA).wait()
        pltpu.make_async_copy(conB, acc_sh.at[sciB], semSB).wait()

        plsc.subcore_barrier()
        pltpu.sync_copy(acc_sh.at[pl.ds(sid * RPT, RPT)],
                        out_hbm.at[cid, pl.ds(sid * RPT, RPT)])

    return edge_kernel


_edge1 = _make_edge_kernel(split=True)
_edge2 = _make_edge_kernel(split=False)


def kernel(x, edge_index, batch, Wl1, bl1, Wr1, br1, att1, b1,
           Wl2, bl2, Wr2, br2, att2, b2, Wlin, blin):
    loop = jnp.arange(N, dtype=edge_index.dtype)
    pad_e = EP - EV
    zpad = jnp.zeros((pad_e,), edge_index.dtype)
    src = jnp.concatenate([edge_index[0], loop, zpad]).reshape(NCHT, 1, K)
    dst = jnp.concatenate([edge_index[1], loop, zpad]).reshape(NCHT, 1, K)
    sd = jnp.concatenate([src, dst], axis=1)
    sd = jnp.pad(sd, ((0, 1), (0, 0), (0, 0)))

    xl1, xr1 = _proj_call(x, Wl1, bl1.reshape(1, -1), Wr1, br1.reshape(1, -1),
                          D_IN, HEADS * D_H)
    acc1 = _edge1(xl1.reshape(2 * N, D_H), xr1.reshape(2 * N, D_H),
                  sd, att1.reshape(-1))
    xl2, xr2 = _mid_call(acc1[0], acc1[1], b1.reshape(1, -1),
                         Wl2, bl2.reshape(1, -1), Wr2, br2.reshape(1, -1))
    acc2 = _edge2(xl2, xr2, sd, att2.reshape(-1))
    batp = batch.reshape(N, 1)
    out = _pool_call(acc2[0], acc2[1], b2.reshape(1, -1), batp,
                     Wlin, blin.reshape(1, -1))
    return out


# speedup vs baseline: 1.0130x; 1.0130x over previous
import functools

import jax
import jax.numpy as jnp
from jax import lax
from jax.experimental import pallas as pl
from jax.experimental.pallas import tpu as pltpu
from jax.experimental.pallas import tpu_sc as plsc

N = 10000
E = 320000
G = 64
D_IN = 128
D_H = 64
HEADS = 2
D_OUT = 64

RB = 400
NBLK = N // RB
NCORES = 2
NSUB = 16
K = 128
EV = E + N
EP = 335872
NCHT = EP // K
RPT = N // NSUB

D = 64
W = 80


def _proj_body(x_ref, wl_ref, bl_ref, wr_ref, br_ref, xl_ref, xr_ref):
    xb = x_ref[...]
    xl_ref[...] = jnp.dot(xb, wl_ref[...], preferred_element_type=jnp.float32) + bl_ref[...]
    xr_ref[...] = jnp.dot(xb, wr_ref[...], preferred_element_type=jnp.float32) + br_ref[...]


def _proj_call(xp, Wl, bl, Wr, br, d_in, d_out):
    return pl.pallas_call(
        _proj_body,
        grid=(NBLK,),
        in_specs=[
            pl.BlockSpec((RB, d_in), lambda i: (i, 0)),
            pl.BlockSpec((d_in, d_out), lambda i: (0, 0)),
            pl.BlockSpec((1, d_out), lambda i: (0, 0)),
            pl.BlockSpec((d_in, d_out), lambda i: (0, 0)),
            pl.BlockSpec((1, d_out), lambda i: (0, 0)),
        ],
        out_specs=[
            pl.BlockSpec((RB, d_out), lambda i: (i, 0)),
            pl.BlockSpec((RB, d_out), lambda i: (i, 0)),
        ],
        out_shape=[
            jax.ShapeDtypeStruct((N, d_out), jnp.float32),
            jax.ShapeDtypeStruct((N, d_out), jnp.float32),
        ],
    )(xp, Wl, bl, Wr, br)


def _mid_body(a0_ref, a1_ref, b1_ref, wl_ref, bl_ref, wr_ref, br_ref,
              xl_ref, xr_ref):
    s0 = a0_ref[...]
    s1 = a1_ref[...]
    h0 = s0[:, 0:64] / (s0[:, 64:65] + 1e-16)
    h1 = s1[:, 0:64] / (s1[:, 64:65] + 1e-16)
    h = jnp.concatenate([h0, h1], axis=1) + b1_ref[...]
    h = jnp.where(h > 0, h, jnp.exp(h) - 1.0)
    xl_ref[...] = jnp.dot(h, wl_ref[...], preferred_element_type=jnp.float32) + bl_ref[...]
    xr_ref[...] = jnp.dot(h, wr_ref[...], preferred_element_type=jnp.float32) + br_ref[...]


def _mid_call(accA, accB, b1, Wl2, bl2, Wr2, br2):
    return pl.pallas_call(
        _mid_body,
        grid=(NBLK,),
        in_specs=[
            pl.BlockSpec((RB, W), lambda i: (i, 0)),
            pl.BlockSpec((RB, W), lambda i: (i, 0)),
            pl.BlockSpec((1, 128), lambda i: (0, 0)),
            pl.BlockSpec((128, 64), lambda i: (0, 0)),
            pl.BlockSpec((1, 64), lambda i: (0, 0)),
            pl.BlockSpec((128, 64), lambda i: (0, 0)),
            pl.BlockSpec((1, 64), lambda i: (0, 0)),
        ],
        out_specs=[
            pl.BlockSpec((RB, 64), lambda i: (i, 0)),
            pl.BlockSpec((RB, 64), lambda i: (i, 0)),
        ],
        out_shape=[
            jax.ShapeDtypeStruct((N, 64), jnp.float32),
            jax.ShapeDtypeStruct((N, 64), jnp.float32),
        ],
    )(accA, accB, b1, Wl2, bl2, Wr2, br2)


def _pool_body(a_ref, b_ref, b2_ref, bat_ref, wlin_ref, blin_ref,
               out_ref, sums_ref, cnt_ref):
    i = pl.program_id(0)
    s = a_ref[...] + b_ref[...]
    h = s[:, 0:64] / (s[:, 64:65] + 1e-16) + b2_ref[...]
    h = jnp.where(h > 0, h, jnp.exp(h) - 1.0)
    gid = lax.broadcasted_iota(jnp.int32, (1, G), 1)
    oh = (bat_ref[...] == gid).astype(jnp.float32)

    @pl.when(i == 0)
    def _():
        sums_ref[...] = jnp.zeros_like(sums_ref)
        cnt_ref[...] = jnp.zeros_like(cnt_ref)

    dn = (((0,), (0,)), ((), ()))
    sums_ref[...] += lax.dot_general(oh, h, dn, preferred_element_type=jnp.float32)
    cnt_ref[...] += lax.dot_general(oh, jnp.ones((RB, 1), jnp.float32), dn,
                                    preferred_element_type=jnp.float32)

    @pl.when(i == NBLK - 1)
    def _():
        pooled = sums_ref[...] / jnp.maximum(cnt_ref[...], 1.0)
        out_ref[...] = jnp.dot(pooled, wlin_ref[...],
                               preferred_element_type=jnp.float32) + blin_ref[...]


def _pool_call(accA, accB, b2, batp, Wlin, blin):
    return pl.pallas_call(
        _pool_body,
        grid=(NBLK,),
        in_specs=[
            pl.BlockSpec((RB, W), lambda i: (i, 0)),
            pl.BlockSpec((RB, W), lambda i: (i, 0)),
            pl.BlockSpec((1, 64), lambda i: (0, 0)),
            pl.BlockSpec((RB, 1), lambda i: (i, 0)),
            pl.BlockSpec((64, 2), lambda i: (0, 0)),
            pl.BlockSpec((1, 2), lambda i: (0, 0)),
        ],
        out_specs=pl.BlockSpec((G, 2), lambda i: (0, 0)),
        out_shape=jax.ShapeDtypeStruct((G, 2), jnp.float32),
        scratch_shapes=[
            pltpu.VMEM((G, 64), jnp.float32),
            pltpu.VMEM((G, 1), jnp.float32),
        ],
    )(accA, accB, b2, batp, Wlin, blin)


def _make_edge_kernel(split):
    NJ = D // 16
    ept = EP // NSUB if split else EP // (NCORES * NSUB)
    nch = ept // K
    mesh = plsc.VectorSubcoreMesh(core_axis_name="c", subcore_axis_name="s")

    @functools.partial(
        pl.kernel,
        mesh=mesh,
        compiler_params=pltpu.CompilerParams(
            needs_layout_passes=False, use_tc_tiling_on_sc=False),
        out_type=jax.ShapeDtypeStruct((NCORES, N, W), jnp.float32),
        scratch_types=[
            pltpu.VMEM((2, K), jnp.int32),
            pltpu.VMEM((2, K), jnp.int32),
            pltpu.VMEM((K,), jnp.int32),
            pltpu.VMEM((K,), jnp.int32),
            pltpu.VMEM((K,), jnp.int32),
            pltpu.VMEM((K,), jnp.int32),
            pltpu.VMEM((K, D), jnp.float32),
            pltpu.VMEM((K, D), jnp.float32),
            pltpu.VMEM((K, D), jnp.float32),
            pltpu.VMEM((K, D), jnp.float32),
            pltpu.VMEM((K,), jnp.int32),
            pltpu.VMEM((K,), jnp.int32),
            pltpu.VMEM((K, W), jnp.float32),
            pltpu.VMEM((K, W), jnp.float32),
            pltpu.VMEM((D * (2 if split else 1),), jnp.float32),
            pltpu.VMEM_SHARED((N, W), jnp.float32),
            pltpu.SemaphoreType.DMA,
            pltpu.SemaphoreType.DMA,
            pltpu.SemaphoreType.DMA,
            pltpu.SemaphoreType.DMA,
            pltpu.SemaphoreType.DMA,
            pltpu.SemaphoreType.DMA,
        ],
    )
    def edge_kernel(xl_hbm, xr_hbm, sd_hbm, att_hbm, out_hbm,
                    sdA, sdB, sgA, dgA, sgB, dgB, xlA, xrA, xlB, xrB,
                    sciA, sciB, conA, conB, att_v, acc_sh,
                    semA1, semA2, semB1, semB2, semSA, semSB):
        cid = lax.axis_index("c")
        sid = lax.axis_index("s")

        zeros16 = jnp.zeros((16,), jnp.float32)

        zi16 = jnp.zeros((16,), jnp.int32)

        def _zrow(r, carry):
            for off in list(range(0, W - 16, 16)) + [W - 16]:
                conA[r, pl.ds(off, 16)] = zeros16
                conB[r, pl.ds(off, 16)] = zeros16
            return carry
        lax.fori_loop(0, K, _zrow, 0)
        for j in range(K // 16):
            sciA[pl.ds(16 * j, 16)] = zi16
            sciB[pl.ds(16 * j, 16)] = zi16
        for t in range(4):
            pltpu.sync_copy(conA, acc_sh.at[pl.ds(sid * RPT + t * K, K)])
        pltpu.sync_copy(conA.at[pl.ds(0, RPT - 4 * K)],
                        acc_sh.at[pl.ds(sid * RPT + 4 * K, RPT - 4 * K)])
        plsc.subcore_barrier()
        pltpu.async_copy(conA, acc_sh.at[sciA], semSA, add=True)
        pltpu.async_copy(conB, acc_sh.at[sciB], semSB, add=True)

        pltpu.sync_copy(att_hbm, att_v)
        aoff = cid * D if split else 0
        attv = [att_v[pl.ds(aoff + 16 * j, 16)] for j in range(NJ)]
        lane = lax.iota(jnp.int32, 16)

        wchunk = (sid if split else cid * NSUB + sid) * nch

        bufs = (
            (sdA, sgA, dgA, xlA, xrA, sciA, conA, semA1, semA2, semSA),
            (sdB, sgB, dgB, xlB, xrB, sciB, conB, semB1, semB2, semSB),
        )

        def _fetch(gch, p):
            sd, sg, dg, xlr, xrr, sci, con, s1, s2, ss = bufs[p]
            pltpu.sync_copy(sd_hbm.at[gch], sd)
            if split:
                for j in range(K // 16):
                    sv = sd[0, pl.ds(16 * j, 16)]
                    dv = sd[1, pl.ds(16 * j, 16)]
                    sg[pl.ds(16 * j, 16)] = sv + sv + cid
                    dg[pl.ds(16 * j, 16)] = dv + dv + cid
                pltpu.async_copy(xl_hbm.at[sg], xlr, s1)
                pltpu.async_copy(xr_hbm.at[dg], xrr, s2)
            else:
                pltpu.async_copy(xl_hbm.at[sd.at[0]], xlr, s1)
                pltpu.async_copy(xr_hbm.at[sd.at[1]], xrr, s2)

        def _wait(p):
            sd, sg, dg, xlr, xrr, sci, con, s1, s2, ss = bufs[p]
            src_ref = xl_hbm.at[sg] if split else xl_hbm.at[sd.at[0]]
            pltpu.make_async_copy(src_ref, xlr, s1).wait()
            src_ref = xr_hbm.at[dg] if split else xr_hbm.at[sd.at[1]]
            pltpu.make_async_copy(src_ref, xrr, s2).wait()

        def _make_edge(xlr, xrr, con):
            def _edge(e, base):
                valid = jnp.full((16,), base + e < EV)
                xlv = [xlr[e, pl.ds(16 * j, 16)] for j in range(NJ)]
                xrv = [xrr[e, pl.ds(16 * j, 16)] for j in range(NJ)]
                pr = []
                for j in range(NJ):
                    z = xlv[j] + xrv[j]
                    pr.append(jnp.maximum(z, 0.2 * z) * attv[j])
                sacc = pr[0]
                for j in range(1, NJ):
                    sacc = sacc + pr[j]
                a = plsc.cumsum(sacc)[15]
                wv = jnp.exp(jnp.full((16,), a, jnp.float32))
                wv = jnp.where(valid, wv, zeros16)
                for j in range(NJ):
                    con[e, pl.ds(16 * j, 16)] = xlv[j] * wv
                con[e, pl.ds(D, 16)] = jnp.where(lane == 0, wv, zeros16)
                return base
            return _edge

        ebase = wchunk * K
        _fetch(wchunk, 0)

        def _pair(g, carry):
            for p in (0, 1):
                ch = 2 * g + p
                _fetch(wchunk + ch + 1, 1 - p)
                _wait(p)
                sd, sg, dg, xlr, xrr, sci, con, s1, s2, ss = bufs[p]
                pltpu.make_async_copy(con, acc_sh.at[sci], ss).wait()
                for j in range(K // 16):
                    sci[pl.ds(16 * j, 16)] = sd[1, pl.ds(16 * j, 16)]
                lax.fori_loop(0, K, _make_edge(xlr, xrr, con),
                              ebase + ch * K, unroll=4)
                pltpu.async_copy(con, acc_sh.at[sci], ss, add=True)
            return carry

        lax.fori_loop(0, nch // 2, _pair, 0)
        _wait(0)
        pltpu.make_async_copy(conA, acc_sh.at[sciA], semSA).wait()
        pltpu.make_async_copy(conB, acc_sh.at[sciB], semSB).wait()

        plsc.subcore_barrier()
        pltpu.sync_copy(acc_sh.at[pl.ds(sid * RPT, RPT)],
                        out_hbm.at[cid, pl.ds(sid * RPT, RPT)])

    return edge_kernel


_edge1 = _make_edge_kernel(split=True)
_edge2 = _make_edge_kernel(split=False)


def kernel(x, edge_index, batch, Wl1, bl1, Wr1, br1, att1, b1,
           Wl2, bl2, Wr2, br2, att2, b2, Wlin, blin):
    loop = jnp.arange(N, dtype=edge_index.dtype)
    pad_e = EP - EV
    zpad = jnp.zeros((pad_e,), edge_index.dtype)
    src = jnp.concatenate([edge_index[0], loop, zpad]).reshape(NCHT, 1, K)
    dst = jnp.concatenate([edge_index[1], loop, zpad]).reshape(NCHT, 1, K)
    sd = jnp.concatenate([src, dst], axis=1)
    sd = jnp.pad(sd, ((0, 1), (0, 0), (0, 0)))

    xl1, xr1 = _proj_call(x, Wl1, bl1.reshape(1, -1), Wr1, br1.reshape(1, -1),
                          D_IN, HEADS * D_H)
    acc1 = _edge1(xl1.reshape(2 * N, D_H), xr1.reshape(2 * N, D_H),
                  sd, att1.reshape(-1))
    xl2, xr2 = _mid_call(acc1[0], acc1[1], b1.reshape(1, -1),
                         Wl2, bl2.reshape(1, -1), Wr2, br2.reshape(1, -1))
    acc2 = _edge2(xl2, xr2, sd, att2.reshape(-1))
    batp = batch.reshape(N, 1)
    out = _pool_call(acc2[0], acc2[1], b2.reshape(1, -1), batp,
                     Wlin, blin.reshape(1, -1))
    return out
